# baseline (speedup 1.0000x reference)
import functools

import jax
import jax.numpy as jnp
from jax import lax
from jax.experimental import pallas as pl
from jax.experimental.pallas import tpu as pltpu
from jax.experimental.pallas import tpu_sc as plsc

MAXLEN = 8192
EMBED_DIM = 128

_info = plsc.get_sparse_core_info()
_NC, _NS = 1, _info.num_subcores
_NW = _NC * _NS
_ROWS_PER_W = MAXLEN // _NW

_mesh = plsc.VectorSubcoreMesh(
    core_axis_name="c", subcore_axis_name="s", num_cores=1
)


_NCHUNK = 4
_CHUNK_ROWS = _ROWS_PER_W // _NCHUNK


@functools.partial(
    pl.kernel,
    mesh=_mesh,
    out_type=jax.ShapeDtypeStruct((MAXLEN, EMBED_DIM), jnp.float32),
    scratch_types=(
        [pltpu.VMEM((_CHUNK_ROWS, EMBED_DIM), jnp.float32)] * _NCHUNK
        + [pltpu.SemaphoreType.DMA] * (2 * _NCHUNK)
    ),
)
def _pos_lookup(table_hbm, out_hbm, *scratch):
    bufs = scratch[:_NCHUNK]
    sems_in = scratch[_NCHUNK : 2 * _NCHUNK]
    sems_out = scratch[2 * _NCHUNK :]
    wid = lax.axis_index("s") * _NC + lax.axis_index("c")
    base = wid * _ROWS_PER_W
    gathers = []
    for i in range(_NCHUNK):
        off = base + i * _CHUNK_ROWS
        gathers.append(
            pltpu.async_copy(
                table_hbm.at[pl.ds(off, _CHUNK_ROWS)], bufs[i], sems_in[i]
            )
        )
    scatters = []
    for i in range(_NCHUNK):
        off = base + i * _CHUNK_ROWS
        gathers[i].wait()
        scatters.append(
            pltpu.async_copy(
                bufs[i], out_hbm.at[pl.ds(off, _CHUNK_ROWS)], sems_out[i]
            )
        )
    for s in scatters:
        s.wait()


def kernel(x, pos_table):
    del x
    return _pos_lookup(pos_table)


# speedup vs baseline: 1.0485x; 1.0485x over previous
import functools

import jax
import jax.numpy as jnp
from jax import lax
from jax.experimental import pallas as pl
from jax.experimental.pallas import tpu as pltpu
from jax.experimental.pallas import tpu_sc as plsc

MAXLEN = 8192
EMBED_DIM = 128

_NC = 2
_ROWS_PER_C = MAXLEN // _NC
_NCHUNK = 4
_CHUNK_ROWS = _ROWS_PER_C // _NCHUNK

_mesh = plsc.ScalarSubcoreMesh(axis_name="c", num_cores=_NC)


@functools.partial(
    pl.kernel,
    mesh=_mesh,
    out_type=jax.ShapeDtypeStruct((MAXLEN, EMBED_DIM), jnp.float32),
    scratch_types=(
        [pltpu.VMEM_SHARED((_CHUNK_ROWS, EMBED_DIM), jnp.float32)] * _NCHUNK
        + [pltpu.SemaphoreType.DMA] * (2 * _NCHUNK)
    ),
)
def _pos_lookup(table_hbm, out_hbm, *scratch):
    bufs = scratch[:_NCHUNK]
    sems_in = scratch[_NCHUNK : 2 * _NCHUNK]
    sems_out = scratch[2 * _NCHUNK :]
    base = lax.axis_index("c") * _ROWS_PER_C
    gathers = []
    for i in range(_NCHUNK):
        off = base + i * _CHUNK_ROWS
        gathers.append(
            pltpu.async_copy(
                table_hbm.at[pl.ds(off, _CHUNK_ROWS)], bufs[i], sems_in[i]
            )
        )
    scatters = []
    for i in range(_NCHUNK):
        off = base + i * _CHUNK_ROWS
        gathers[i].wait()
        scatters.append(
            pltpu.async_copy(
                bufs[i], out_hbm.at[pl.ds(off, _CHUNK_ROWS)], sems_out[i]
            )
        )
    for s in scatters:
        s.wait()


def kernel(x, pos_table):
    del x
    return _pos_lookup(pos_table)


# speedup vs baseline: 1.0530x; 1.0043x over previous
import functools

import jax
import jax.numpy as jnp
from jax import lax
from jax.experimental import pallas as pl
from jax.experimental.pallas import tpu as pltpu
from jax.experimental.pallas import tpu_sc as plsc

MAXLEN = 8192
EMBED_DIM = 128

_info = plsc.get_sparse_core_info()
_NC, _NS = _info.num_cores, _info.num_subcores
_NW = _NC * _NS
_ROWS_PER_W = MAXLEN // _NW

_mesh = plsc.VectorSubcoreMesh(core_axis_name="c", subcore_axis_name="s")


_NCHUNK = 2
_CHUNK_ROWS = _ROWS_PER_W // _NCHUNK


@functools.partial(
    pl.kernel,
    mesh=_mesh,
    out_type=jax.ShapeDtypeStruct((MAXLEN, EMBED_DIM), jnp.float32),
    scratch_types=(
        [pltpu.VMEM((_CHUNK_ROWS, EMBED_DIM), jnp.float32)] * _NCHUNK
        + [pltpu.SemaphoreType.DMA] * (2 * _NCHUNK)
    ),
)
def _pos_lookup(table_hbm, out_hbm, *scratch):
    bufs = scratch[:_NCHUNK]
    sems_in = scratch[_NCHUNK : 2 * _NCHUNK]
    sems_out = scratch[2 * _NCHUNK :]
    wid = lax.axis_index("s") * _NC + lax.axis_index("c")
    base = wid * _ROWS_PER_W
    gathers = []
    for i in range(_NCHUNK):
        off = base + i * _CHUNK_ROWS
        gathers.append(
            pltpu.async_copy(
                table_hbm.at[pl.ds(off, _CHUNK_ROWS)], bufs[i], sems_in[i]
            )
        )
    scatters = []
    for i in range(_NCHUNK):
        off = base + i * _CHUNK_ROWS
        gathers[i].wait()
        scatters.append(
            pltpu.async_copy(
                bufs[i], out_hbm.at[pl.ds(off, _CHUNK_ROWS)], sems_out[i]
            )
        )
    for s in scatters:
        s.wait()


def kernel(x, pos_table):
    del x
    return _pos_lookup(pos_table)
